# Initial kernel scaffold; baseline (speedup 1.0000x reference)
import jax
import jax.numpy as jnp
from jax.experimental import pallas as pl


def kernel(r):
    raise NotImplementedError("write your pallas kernel here")


# speedup vs baseline: 34.6755x; 34.6755x over previous
import jax
import jax.numpy as jnp
from jax.experimental import pallas as pl

NXK = 256
ROWS = NXK * NXK
BLOCK_ROWS = 4096


def _zero_last_lane_kernel(x_ref, o_ref):
    x = x_ref[...]
    lane = jax.lax.broadcasted_iota(jnp.int32, x.shape, dimension=1)
    o_ref[...] = jnp.where(lane == NXK - 1, jnp.float32(0), x)


def kernel(r):
    flat = r.reshape(ROWS, NXK)
    out = pl.pallas_call(
        _zero_last_lane_kernel,
        grid=(ROWS // BLOCK_ROWS,),
        in_specs=[pl.BlockSpec((BLOCK_ROWS, NXK), lambda i: (i, 0))],
        out_specs=pl.BlockSpec((BLOCK_ROWS, NXK), lambda i: (i, 0)),
        out_shape=jax.ShapeDtypeStruct((ROWS, NXK), jnp.float32),
    )(flat)
    return out.reshape(r.shape)


# speedup vs baseline: 34.9784x; 1.0087x over previous
import jax
import jax.numpy as jnp
from jax.experimental import pallas as pl

NXK = 256
ROWS = NXK * NXK
BLOCK_ROWS = 8192


def _zero_last_lane_kernel(x_ref, o_ref):
    x = x_ref[...]
    lane = jax.lax.broadcasted_iota(jnp.int32, (1, NXK), 1)
    o_ref[...] = jnp.where(lane != NXK - 1, x, jnp.float32(0))


def kernel(r):
    flat = r.reshape(ROWS, NXK)
    out = pl.pallas_call(
        _zero_last_lane_kernel,
        grid=(ROWS // BLOCK_ROWS,),
        in_specs=[pl.BlockSpec((BLOCK_ROWS, NXK), lambda i: (i, 0))],
        out_specs=pl.BlockSpec((BLOCK_ROWS, NXK), lambda i: (i, 0)),
        out_shape=jax.ShapeDtypeStruct((ROWS, NXK), jnp.float32),
    )(flat)
    return out.reshape(r.shape)


# speedup vs baseline: 34.9907x; 1.0004x over previous
import jax
import jax.numpy as jnp
from jax.experimental import pallas as pl
from jax.experimental.pallas import tpu as pltpu

NXK = 256
ROWS = NXK * NXK
CH_ROWS = 2048
NCHUNKS = ROWS // CH_ROWS
NBUF = 4


def _copy_zero_kernel(x_hbm, o_hbm, in_buf, out_buf, in_sems, out_sems):
    lane = jax.lax.broadcasted_iota(jnp.int32, (1, NXK), 1)
    keep = lane != NXK - 1

    def in_copy(i):
        return pltpu.make_async_copy(
            x_hbm.at[pl.ds(i * CH_ROWS, CH_ROWS), :],
            in_buf.at[i % NBUF],
            in_sems.at[i % NBUF],
        )

    def out_copy(i):
        return pltpu.make_async_copy(
            out_buf.at[i % NBUF],
            o_hbm.at[pl.ds(i * CH_ROWS, CH_ROWS), :],
            out_sems.at[i % NBUF],
        )

    for i in range(min(NBUF, NCHUNKS)):
        in_copy(i).start()
    for i in range(NCHUNKS):
        if i >= NBUF:
            out_copy(i - NBUF).wait()
        in_copy(i).wait()
        out_buf[i % NBUF] = jnp.where(keep, in_buf[i % NBUF], jnp.float32(0))
        out_copy(i).start()
        if i + NBUF < NCHUNKS:
            in_copy(i + NBUF).start()
    for i in range(max(0, NCHUNKS - NBUF), NCHUNKS):
        out_copy(i).wait()


def kernel(r):
    flat = r.reshape(ROWS, NXK)
    out = pl.pallas_call(
        _copy_zero_kernel,
        in_specs=[pl.BlockSpec(memory_space=pltpu.MemorySpace.HBM)],
        out_specs=pl.BlockSpec(memory_space=pltpu.MemorySpace.HBM),
        out_shape=jax.ShapeDtypeStruct((ROWS, NXK), jnp.float32),
        scratch_shapes=[
            pltpu.VMEM((NBUF, CH_ROWS, NXK), jnp.float32),
            pltpu.VMEM((NBUF, CH_ROWS, NXK), jnp.float32),
            pltpu.SemaphoreType.DMA((NBUF,)),
            pltpu.SemaphoreType.DMA((NBUF,)),
        ],
    )(flat)
    return out.reshape(r.shape)


# speedup vs baseline: 80.4341x; 2.2987x over previous
import functools

import jax
import jax.numpy as jnp
from jax import lax
from jax.experimental import pallas as pl
from jax.experimental.pallas import tpu as pltpu
from jax.experimental.pallas import tpu_sc as plsc

NXK = 256
ROWS = NXK * NXK
NC, NS = 2, 16
NW = NC * NS
RPW = ROWS // NW
CH = 128
NCHUNK = RPW // CH
NBUF = 3

_MESH = plsc.VectorSubcoreMesh(core_axis_name="c", subcore_axis_name="s")


@functools.partial(
    pl.kernel,
    out_type=jax.ShapeDtypeStruct((ROWS, NXK), jnp.float32),
    mesh=_MESH,
    scratch_types=[
        pltpu.VMEM((NBUF, CH, NXK), jnp.float32),
        pltpu.SemaphoreType.DMA((NBUF,)),
        pltpu.SemaphoreType.DMA((NBUF,)),
    ],
    compiler_params=pltpu.CompilerParams(
        use_tc_tiling_on_sc=False, needs_layout_passes=False
    ),
)
def _sc_copy_zero(x_hbm, o_hbm, buf, in_sems, out_sems):
    wid = lax.axis_index("s") * NC + lax.axis_index("c")
    base = wid * RPW

    def in_copy(i):
        return pltpu.make_async_copy(
            x_hbm.at[pl.ds(base + i * CH, CH), :],
            buf.at[i % NBUF],
            in_sems.at[i % NBUF],
        )

    def out_copy(i):
        return pltpu.make_async_copy(
            buf.at[i % NBUF],
            o_hbm.at[pl.ds(base + i * CH, CH), :],
            out_sems.at[i % NBUF],
        )

    lane = lax.iota(jnp.int32, 16)
    col = jnp.full((16,), NXK - 1, jnp.int32)
    zeros = jnp.zeros((16,), jnp.float32)

    for i in range(min(NBUF - 1, NCHUNK)):
        in_copy(i).start()
    for i in range(NCHUNK):
        in_copy(i).wait()
        s = jnp.full((16,), i % NBUF, jnp.int32)
        for j in range(CH // 16):
            plsc.store_scatter(buf, [s, lane + 16 * j, col], zeros)
        out_copy(i).start()
        if i + NBUF - 1 < NCHUNK:
            if i >= 1:
                out_copy(i - 1).wait()
            in_copy(i + NBUF - 1).start()
    for i in range(max(0, NCHUNK - NBUF), NCHUNK):
        out_copy(i).wait()


def kernel(r):
    flat = r.reshape(ROWS, NXK)
    return _sc_copy_zero(flat).reshape(r.shape)
